# baseline (speedup 1.0000x reference)
import functools

import jax
import jax.numpy as jnp
from jax import lax
from jax.experimental import pallas as pl
from jax.experimental.pallas import tpu as pltpu
from jax.experimental.pallas import tpu_sc as plsc

NSAMPLE = 16
_KNN_T = 1024
_MLP_T = 512
_PROJ_T = 2048


def _knn_body(q_ref, k_ref, o_ref, *, n, nsample):
    x = q_ref[0]
    y = k_ref[0]
    t = x.shape[0]
    dot = lax.dot_general(x.astype(jnp.bfloat16), y.astype(jnp.bfloat16),
                          (((1,), (0,)), ((), ())),
                          preferred_element_type=jnp.float32)
    n1 = (x[:, 0:1] * x[:, 0:1] + x[:, 1:2] * x[:, 1:2]) + x[:, 2:3] * x[:, 2:3]
    n2 = (y[0:1, :] * y[0:1, :] + y[1:2, :] * y[1:2, :]) + y[2:3, :] * y[2:3, :]
    d = (-2.0 * dot + n1) + n2
    kd = lax.bitcast_convert_type(jnp.maximum(d, 0.0), jnp.int32)
    iota = lax.broadcasted_iota(jnp.int32, (t, n), 1)
    key = jnp.bitwise_or(jnp.bitwise_and(kd, -4096), iota)
    cols = []
    for _ in range(nsample - 2):
        m = jnp.min(key, axis=1, keepdims=True)
        cols.append(jnp.bitwise_and(m, 4095))
        key = jnp.where(key == m, jnp.int32(2147483647), key)
    d2 = jnp.where(key == jnp.int32(2147483647), jnp.inf, d)
    for _ in range(2):
        m = jnp.min(d2, axis=1, keepdims=True)
        cand = jnp.where(d2 == m, iota, n)
        idxj = jnp.min(cand, axis=1, keepdims=True)
        cols.append(idxj)
        d2 = jnp.where(iota == idxj, jnp.inf, d2)
    o_ref[0] = jnp.concatenate(cols, axis=1)


def _knn(qpts, kpts, nsample):
    g, n, _ = qpts.shape
    t = _KNN_T
    return pl.pallas_call(
        functools.partial(_knn_body, n=n, nsample=nsample),
        grid=(g, n // t),
        in_specs=[
            pl.BlockSpec((1, t, 3), lambda d, i: (d, i, 0)),
            pl.BlockSpec((1, 3, n), lambda d, i: (d, 0, 0)),
        ],
        out_specs=pl.BlockSpec((1, t, nsample), lambda d, i: (d, i, 0)),
        out_shape=jax.ShapeDtypeStruct((g, n, nsample), jnp.int32),
    )(qpts, kpts)


def _proj_body(p_ref, x_ref, w_ref, o_ref, *, d_in):
    w = w_ref[...]
    wb = w[:, d_in:2 * d_in]
    wc = w[:, 2 * d_in:]
    s = (lax.dot_general(p_ref[...], wb, (((1,), (1,)), ((), ())),
                         preferred_element_type=jnp.float32)
         + lax.dot_general(x_ref[...], wc, (((1,), (1,)), ((), ())),
                           preferred_element_type=jnp.float32))
    o_ref[...] = s


def _proj(pts, xyz, w):
    m, d_in = pts.shape
    cout = w.shape[0]
    t = _PROJ_T
    return pl.pallas_call(
        functools.partial(_proj_body, d_in=d_in),
        grid=(m // t,),
        in_specs=[
            pl.BlockSpec((t, d_in), lambda i: (i, 0)),
            pl.BlockSpec((t, 3), lambda i: (i, 0)),
            pl.BlockSpec(w.shape, lambda i: (0, 0)),
        ],
        out_specs=pl.BlockSpec((t, cout), lambda i: (i, 0)),
        out_shape=jax.ShapeDtypeStruct((m, cout), jnp.float32),
    )(pts, xyz, w)


def _gather_rows(table, idx):
    v, d = table.shape
    (m,) = idx.shape
    info = plsc.get_sparse_core_info()
    nc, ns = info.num_cores, info.num_subcores
    nw = nc * ns
    b_per_w = m // nw
    chunk = min(b_per_w, 128)

    @functools.partial(
        pl.kernel,
        out_type=jax.ShapeDtypeStruct((m, d), table.dtype),
        mesh=plsc.VectorSubcoreMesh(core_axis_name="c", subcore_axis_name="s"),
        scratch_types=[
            pltpu.VMEM((b_per_w,), jnp.int32),
            pltpu.VMEM((chunk, d), table.dtype),
            pltpu.VMEM((chunk, d), table.dtype),
            pltpu.SemaphoreType.DMA,
            pltpu.SemaphoreType.DMA,
            pltpu.SemaphoreType.DMA,
            pltpu.SemaphoreType.DMA,
        ],
    )
    def gk(table_hbm, idx_hbm, out_hbm, idx_v, rows_a, rows_b,
           sem_a, sem_b, osem_a, osem_b):
        wid = lax.axis_index("s") * nc + lax.axis_index("c")
        base = wid * b_per_w
        pltpu.sync_copy(idx_hbm.at[pl.ds(base, b_per_w)], idx_v)

        def body(i, carry):
            c0 = i * (2 * chunk)
            ga = pltpu.async_copy(
                table_hbm.at[idx_v.at[pl.ds(c0, chunk)]], rows_a, sem_a)
            gb = pltpu.async_copy(
                table_hbm.at[idx_v.at[pl.ds(c0 + chunk, chunk)]], rows_b, sem_b)
            ga.wait()
            wa = pltpu.async_copy(rows_a, out_hbm.at[pl.ds(base + c0, chunk)],
                                  osem_a)
            gb.wait()
            wb = pltpu.async_copy(rows_b,
                                  out_hbm.at[pl.ds(base + c0 + chunk, chunk)],
                                  osem_b)
            wa.wait()
            wb.wait()
            return carry

        lax.fori_loop(0, b_per_w // (2 * chunk), body, 0)

    return gk(table, idx)


def _leaky(x):
    return jnp.where(x > 0, x, 0.1 * x)


def _mlp_body(*refs, d_in, nlayers, nsample):
    p_ref, x_ref, g_ref = refs[0], refs[1], refs[2]
    w0_ref, b0_ref = refs[3], refs[4]
    lrefs = refs[5:5 + 2 * nlayers]
    o_ref = refs[5 + 2 * nlayers]
    w0 = w0_ref[...]
    wa = w0[:, :d_in]
    wc = w0[:, 2 * d_in:]
    q = (lax.dot_general(p_ref[...], wa, (((1,), (1,)), ((), ())),
                         preferred_element_type=jnp.float32)
         - lax.dot_general(x_ref[...], wc, (((1,), (1,)), ((), ())),
                           preferred_element_type=jnp.float32)
         + b0_ref[...])
    t, cout = q.shape
    g = g_ref[...].astype(jnp.float32)
    h3 = g.reshape(t, nsample, cout) + q[:, None, :]
    h = _leaky(h3).reshape(t * nsample, cout)
    for li in range(nlayers):
        w = lrefs[2 * li][...].astype(jnp.bfloat16)
        b = lrefs[2 * li + 1][...]
        h = _leaky(lax.dot_general(h.astype(jnp.bfloat16), w,
                                   (((1,), (1,)), ((), ())),
                                   preferred_element_type=jnp.float32) + b)
    o_ref[...] = jnp.max(h.reshape(t, nsample, cout), axis=1)


def _mlp(pts_q, xyz_q, g, w0, b0, layers, nsample):
    m, d_in = pts_q.shape
    cout = w0.shape[0]
    t = _MLP_T
    nlayers = len(layers)
    operands = [pts_q, xyz_q, g, w0, b0.reshape(1, cout)]
    in_specs = [
        pl.BlockSpec((t, d_in), lambda i: (i, 0)),
        pl.BlockSpec((t, 3), lambda i: (i, 0)),
        pl.BlockSpec((t * nsample, cout), lambda i: (i, 0)),
        pl.BlockSpec(w0.shape, lambda i: (0, 0)),
        pl.BlockSpec((1, cout), lambda i: (0, 0)),
    ]
    for w, b in layers:
        operands += [w, b.reshape(1, cout)]
        in_specs += [pl.BlockSpec(w.shape, lambda i: (0, 0)),
                     pl.BlockSpec((1, cout), lambda i: (0, 0))]
    return pl.pallas_call(
        functools.partial(_mlp_body, d_in=d_in, nlayers=nlayers,
                          nsample=nsample),
        grid=(m // t,),
        in_specs=in_specs,
        out_specs=pl.BlockSpec((t, cout), lambda i: (i, 0)),
        out_shape=jax.ShapeDtypeStruct((m, cout), jnp.float32),
    )(*operands)


def kernel(pc1, pc2, feat1, feat2, W1_0, b1_0, W1_1, b1_1, W1_2, b1_2,
           W2_0, b2_0, W2_1, b2_1):
    b, _, n = pc1.shape
    c = feat1.shape[1]
    k = NSAMPLE
    p1 = jnp.transpose(pc1, (0, 2, 1))
    p2 = jnp.transpose(pc2, (0, 2, 1))
    f1 = jnp.transpose(feat1, (0, 2, 1))
    f2 = jnp.transpose(feat2, (0, 2, 1))

    spts = jnp.concatenate([f2, f1], 0).reshape(2 * b * n, c)
    sxyz = jnp.concatenate([p2, p1], 0).reshape(2 * b * n, 3)
    s12 = _proj(spts, sxyz, W1_0)

    offs = (jnp.arange(b, dtype=jnp.int32) * n)[:, None, None]
    idx1 = _knn(p2, pc1, k)
    g1 = _gather_rows(s12, (idx1 + offs + b * n).reshape(-1))
    idx0 = _knn(p1, pc2, k)
    g0 = _gather_rows(s12, (idx0 + offs).reshape(-1))
    out1 = _mlp(f2.reshape(b * n, c), p2.reshape(b * n, 3), g1,
                W1_0, b1_0, [(W1_1, b1_1), (W1_2, b1_2)], k)
    feat2_new = out1.reshape(b, n, -1)

    cn = feat2_new.shape[-1]
    s3 = _proj(feat2_new.reshape(b * n, cn), p2.reshape(b * n, 3), W2_0)
    g3 = _gather_rows(s3, (idx0 + offs).reshape(-1))
    out0 = _mlp(f1.reshape(b * n, c), p1.reshape(b * n, 3), g0,
                W1_0, b1_0, [(W1_1, b1_1), (W1_2, b1_2)], k)
    feat1_new = out0.reshape(b, n, -1)
    out3 = _mlp(feat1_new.reshape(b * n, cn), p1.reshape(b * n, 3), g3,
                W2_0, b2_0, [(W2_1, b2_1)], k)

    to_cn = lambda x: jnp.transpose(x, (0, 2, 1))
    return (to_cn(feat1_new), to_cn(feat2_new),
            to_cn(out3.reshape(b, n, -1)))


# speedup vs baseline: 1.1586x; 1.1586x over previous
import functools

import jax
import jax.numpy as jnp
from jax import lax
from jax.experimental import pallas as pl
from jax.experimental.pallas import tpu as pltpu
from jax.experimental.pallas import tpu_sc as plsc

NSAMPLE = 16
_KNN_T = 512
_MLP_T = 512
_PROJ_T = 2048


def _knn_body(q_ref, k_ref, o_ref, *, n, nsample):
    x = q_ref[0]
    y = k_ref[0]
    t = x.shape[0]
    dot = lax.dot_general(x.astype(jnp.bfloat16), y.astype(jnp.bfloat16),
                          (((1,), (0,)), ((), ())),
                          preferred_element_type=jnp.float32)
    n1 = (x[:, 0:1] * x[:, 0:1] + x[:, 1:2] * x[:, 1:2]) + x[:, 2:3] * x[:, 2:3]
    n2 = (y[0:1, :] * y[0:1, :] + y[1:2, :] * y[1:2, :]) + y[2:3, :] * y[2:3, :]
    d = (-2.0 * dot + n1) + n2
    kd = lax.bitcast_convert_type(jnp.maximum(d, 0.0), jnp.int32)
    iota = lax.broadcasted_iota(jnp.int32, (t, n), 1)
    key = jnp.bitwise_or(jnp.bitwise_and(kd, -4096), iota)
    cols = []
    for _ in range(nsample - 2):
        m = jnp.min(key, axis=1, keepdims=True)
        cols.append(jnp.bitwise_and(m, 4095))
        key = jnp.where(key == m, jnp.int32(2147483647), key)
    d2 = jnp.where(key == jnp.int32(2147483647), jnp.inf, d)
    for _ in range(2):
        m = jnp.min(d2, axis=1, keepdims=True)
        cand = jnp.where(d2 == m, iota, n)
        idxj = jnp.min(cand, axis=1, keepdims=True)
        cols.append(idxj)
        d2 = jnp.where(iota == idxj, jnp.inf, d2)
    o_ref[0] = jnp.concatenate(cols, axis=1)


def _knn(qpts, kpts, nsample):
    g, n, _ = qpts.shape
    t = _KNN_T
    return pl.pallas_call(
        functools.partial(_knn_body, n=n, nsample=nsample),
        grid=(g, n // t),
        in_specs=[
            pl.BlockSpec((1, t, 3), lambda d, i: (d, i, 0)),
            pl.BlockSpec((1, 3, n), lambda d, i: (d, 0, 0)),
        ],
        out_specs=pl.BlockSpec((1, t, nsample), lambda d, i: (d, i, 0)),
        out_shape=jax.ShapeDtypeStruct((g, n, nsample), jnp.int32),
    )(qpts, kpts)


def _proj_body(p_ref, x_ref, w_ref, o_ref, *, d_in):
    w = w_ref[...]
    wb = w[:, d_in:2 * d_in]
    wc = w[:, 2 * d_in:]
    s = (lax.dot_general(p_ref[...], wb, (((1,), (1,)), ((), ())),
                         preferred_element_type=jnp.float32)
         + lax.dot_general(x_ref[...], wc, (((1,), (1,)), ((), ())),
                           preferred_element_type=jnp.float32))
    o_ref[...] = s


def _proj(pts, xyz, w):
    m, d_in = pts.shape
    cout = w.shape[0]
    t = _PROJ_T
    return pl.pallas_call(
        functools.partial(_proj_body, d_in=d_in),
        grid=(m // t,),
        in_specs=[
            pl.BlockSpec((t, d_in), lambda i: (i, 0)),
            pl.BlockSpec((t, 3), lambda i: (i, 0)),
            pl.BlockSpec(w.shape, lambda i: (0, 0)),
        ],
        out_specs=pl.BlockSpec((t, cout), lambda i: (i, 0)),
        out_shape=jax.ShapeDtypeStruct((m, cout), jnp.float32),
    )(pts, xyz, w)


def _gather_rows(table, idx):
    v, d = table.shape
    (m,) = idx.shape
    info = plsc.get_sparse_core_info()
    nc, ns = info.num_cores, info.num_subcores
    nw = nc * ns
    b_per_w = m // nw
    chunk = min(b_per_w, 128)

    @functools.partial(
        pl.kernel,
        out_type=jax.ShapeDtypeStruct((m, d), table.dtype),
        mesh=plsc.VectorSubcoreMesh(core_axis_name="c", subcore_axis_name="s"),
        scratch_types=[
            pltpu.VMEM((b_per_w,), jnp.int32),
            pltpu.VMEM((chunk, d), table.dtype),
            pltpu.VMEM((chunk, d), table.dtype),
            pltpu.SemaphoreType.DMA,
            pltpu.SemaphoreType.DMA,
            pltpu.SemaphoreType.DMA,
            pltpu.SemaphoreType.DMA,
        ],
    )
    def gk(table_hbm, idx_hbm, out_hbm, idx_v, rows_a, rows_b,
           sem_a, sem_b, osem_a, osem_b):
        wid = lax.axis_index("s") * nc + lax.axis_index("c")
        base = wid * b_per_w
        pltpu.sync_copy(idx_hbm.at[pl.ds(base, b_per_w)], idx_v)

        def body(i, carry):
            c0 = i * (2 * chunk)
            ga = pltpu.async_copy(
                table_hbm.at[idx_v.at[pl.ds(c0, chunk)]], rows_a, sem_a)
            gb = pltpu.async_copy(
                table_hbm.at[idx_v.at[pl.ds(c0 + chunk, chunk)]], rows_b, sem_b)
            ga.wait()
            wa = pltpu.async_copy(rows_a, out_hbm.at[pl.ds(base + c0, chunk)],
                                  osem_a)
            gb.wait()
            wb = pltpu.async_copy(rows_b,
                                  out_hbm.at[pl.ds(base + c0 + chunk, chunk)],
                                  osem_b)
            wa.wait()
            wb.wait()
            return carry

        lax.fori_loop(0, b_per_w // (2 * chunk), body, 0)

    return gk(table, idx)


def _leaky(x):
    return jnp.where(x > 0, x, 0.1 * x)


def _mlp_body(*refs, d_in, nlayers, nsample):
    p_ref, x_ref, g_ref = refs[0], refs[1], refs[2]
    w0_ref, b0_ref = refs[3], refs[4]
    lrefs = refs[5:5 + 2 * nlayers]
    o_ref = refs[5 + 2 * nlayers]
    w0 = w0_ref[...]
    wa = w0[:, :d_in]
    wc = w0[:, 2 * d_in:]
    q = (lax.dot_general(p_ref[...], wa, (((1,), (1,)), ((), ())),
                         preferred_element_type=jnp.float32)
         - lax.dot_general(x_ref[...], wc, (((1,), (1,)), ((), ())),
                           preferred_element_type=jnp.float32)
         + b0_ref[...])
    t, cout = q.shape
    g = g_ref[...].astype(jnp.float32)
    h3 = g.reshape(t, nsample, cout) + q[:, None, :]
    h = _leaky(h3).reshape(t * nsample, cout)
    for li in range(nlayers):
        w = lrefs[2 * li][...].astype(jnp.bfloat16)
        b = lrefs[2 * li + 1][...]
        h = _leaky(lax.dot_general(h.astype(jnp.bfloat16), w,
                                   (((1,), (1,)), ((), ())),
                                   preferred_element_type=jnp.float32) + b)
    o_ref[...] = jnp.max(h.reshape(t, nsample, cout), axis=1)


def _mlp(pts_q, xyz_q, g, w0, b0, layers, nsample):
    m, d_in = pts_q.shape
    cout = w0.shape[0]
    t = _MLP_T
    nlayers = len(layers)
    operands = [pts_q, xyz_q, g, w0, b0.reshape(1, cout)]
    in_specs = [
        pl.BlockSpec((t, d_in), lambda i: (i, 0)),
        pl.BlockSpec((t, 3), lambda i: (i, 0)),
        pl.BlockSpec((t * nsample, cout), lambda i: (i, 0)),
        pl.BlockSpec(w0.shape, lambda i: (0, 0)),
        pl.BlockSpec((1, cout), lambda i: (0, 0)),
    ]
    for w, b in layers:
        operands += [w, b.reshape(1, cout)]
        in_specs += [pl.BlockSpec(w.shape, lambda i: (0, 0)),
                     pl.BlockSpec((1, cout), lambda i: (0, 0))]
    return pl.pallas_call(
        functools.partial(_mlp_body, d_in=d_in, nlayers=nlayers,
                          nsample=nsample),
        grid=(m // t,),
        in_specs=in_specs,
        out_specs=pl.BlockSpec((t, cout), lambda i: (i, 0)),
        out_shape=jax.ShapeDtypeStruct((m, cout), jnp.float32),
    )(*operands)


def kernel(pc1, pc2, feat1, feat2, W1_0, b1_0, W1_1, b1_1, W1_2, b1_2,
           W2_0, b2_0, W2_1, b2_1):
    b, _, n = pc1.shape
    c = feat1.shape[1]
    k = NSAMPLE
    p1 = jnp.transpose(pc1, (0, 2, 1))
    p2 = jnp.transpose(pc2, (0, 2, 1))
    f1 = jnp.transpose(feat1, (0, 2, 1))
    f2 = jnp.transpose(feat2, (0, 2, 1))

    spts = jnp.concatenate([f2, f1], 0).reshape(2 * b * n, c)
    sxyz = jnp.concatenate([p2, p1], 0).reshape(2 * b * n, 3)
    s12 = _proj(spts, sxyz, W1_0)

    offs = (jnp.arange(b, dtype=jnp.int32) * n)[:, None, None]
    idx1 = _knn(p2, pc1, k)
    g1 = _gather_rows(s12, (idx1 + offs + b * n).reshape(-1))
    idx0 = _knn(p1, pc2, k)
    g0 = _gather_rows(s12, (idx0 + offs).reshape(-1))
    out1 = _mlp(f2.reshape(b * n, c), p2.reshape(b * n, 3), g1,
                W1_0, b1_0, [(W1_1, b1_1), (W1_2, b1_2)], k)
    feat2_new = out1.reshape(b, n, -1)

    cn = feat2_new.shape[-1]
    s3 = _proj(feat2_new.reshape(b * n, cn), p2.reshape(b * n, 3), W2_0)
    g3 = _gather_rows(s3, (idx0 + offs).reshape(-1))
    out0 = _mlp(f1.reshape(b * n, c), p1.reshape(b * n, 3), g0,
                W1_0, b1_0, [(W1_1, b1_1), (W1_2, b1_2)], k)
    feat1_new = out0.reshape(b, n, -1)
    out3 = _mlp(feat1_new.reshape(b * n, cn), p1.reshape(b * n, 3), g3,
                W2_0, b2_0, [(W2_1, b2_1)], k)

    to_cn = lambda x: jnp.transpose(x, (0, 2, 1))
    return (to_cn(feat1_new), to_cn(feat2_new),
            to_cn(out3.reshape(b, n, -1)))


# speedup vs baseline: 1.1843x; 1.0221x over previous
import functools

import jax
import jax.numpy as jnp
from jax import lax
from jax.experimental import pallas as pl
from jax.experimental.pallas import tpu as pltpu
from jax.experimental.pallas import tpu_sc as plsc

NSAMPLE = 16
_KNN_T = 512
_MLP_T = 512
_PROJ_T = 2048


def _knn_body(q_ref, k_ref, o_ref, *, n, nsample):
    x = q_ref[0]
    y = k_ref[0]
    t = x.shape[0]
    dot = lax.dot_general(x.astype(jnp.bfloat16), y.astype(jnp.bfloat16),
                          (((1,), (0,)), ((), ())),
                          preferred_element_type=jnp.float32)
    n1 = (x[:, 0:1] * x[:, 0:1] + x[:, 1:2] * x[:, 1:2]) + x[:, 2:3] * x[:, 2:3]
    n2 = (y[0:1, :] * y[0:1, :] + y[1:2, :] * y[1:2, :]) + y[2:3, :] * y[2:3, :]
    d = (-2.0 * dot + n1) + n2
    kd = lax.bitcast_convert_type(jnp.maximum(d, 0.0), jnp.int32)
    iota = lax.broadcasted_iota(jnp.int32, (t, n), 1)
    key = jnp.bitwise_or(jnp.bitwise_and(kd, -4096), iota)
    cols = []
    for _ in range(nsample - 1):
        m = jnp.min(key, axis=1, keepdims=True)
        cols.append(jnp.bitwise_and(m, 4095))
        key = jnp.where(key == m, jnp.int32(2147483647), key)
    d2 = jnp.where(key == jnp.int32(2147483647), jnp.inf, d)
    for _ in range(1):
        m = jnp.min(d2, axis=1, keepdims=True)
        cand = jnp.where(d2 == m, iota, n)
        idxj = jnp.min(cand, axis=1, keepdims=True)
        cols.append(idxj)
        d2 = jnp.where(iota == idxj, jnp.inf, d2)
    o_ref[0] = jnp.concatenate(cols, axis=1)


def _knn(qpts, kpts, nsample):
    g, n, _ = qpts.shape
    t = _KNN_T
    return pl.pallas_call(
        functools.partial(_knn_body, n=n, nsample=nsample),
        grid=(g, n // t),
        in_specs=[
            pl.BlockSpec((1, t, 3), lambda d, i: (d, i, 0)),
            pl.BlockSpec((1, 3, n), lambda d, i: (d, 0, 0)),
        ],
        out_specs=pl.BlockSpec((1, t, nsample), lambda d, i: (d, i, 0)),
        out_shape=jax.ShapeDtypeStruct((g, n, nsample), jnp.int32),
    )(qpts, kpts)


def _proj_body(p_ref, x_ref, w_ref, o_ref, *, d_in):
    w = w_ref[...]
    wb = w[:, d_in:2 * d_in]
    wc = w[:, 2 * d_in:]
    s = (lax.dot_general(p_ref[...], wb, (((1,), (1,)), ((), ())),
                         preferred_element_type=jnp.float32)
         + lax.dot_general(x_ref[...], wc, (((1,), (1,)), ((), ())),
                           preferred_element_type=jnp.float32))
    o_ref[...] = s


def _proj(pts, xyz, w):
    m, d_in = pts.shape
    cout = w.shape[0]
    t = _PROJ_T
    return pl.pallas_call(
        functools.partial(_proj_body, d_in=d_in),
        grid=(m // t,),
        in_specs=[
            pl.BlockSpec((t, d_in), lambda i: (i, 0)),
            pl.BlockSpec((t, 3), lambda i: (i, 0)),
            pl.BlockSpec(w.shape, lambda i: (0, 0)),
        ],
        out_specs=pl.BlockSpec((t, cout), lambda i: (i, 0)),
        out_shape=jax.ShapeDtypeStruct((m, cout), jnp.float32),
    )(pts, xyz, w)


def _gather_rows(table, idx):
    v, d = table.shape
    (m,) = idx.shape
    info = plsc.get_sparse_core_info()
    nc, ns = info.num_cores, info.num_subcores
    nw = nc * ns
    b_per_w = m // nw
    chunk = min(b_per_w, 128)

    @functools.partial(
        pl.kernel,
        out_type=jax.ShapeDtypeStruct((m, d), table.dtype),
        mesh=plsc.VectorSubcoreMesh(core_axis_name="c", subcore_axis_name="s"),
        scratch_types=[
            pltpu.VMEM((b_per_w,), jnp.int32),
            pltpu.VMEM((chunk, d), table.dtype),
            pltpu.VMEM((chunk, d), table.dtype),
            pltpu.SemaphoreType.DMA,
            pltpu.SemaphoreType.DMA,
            pltpu.SemaphoreType.DMA,
            pltpu.SemaphoreType.DMA,
        ],
    )
    def gk(table_hbm, idx_hbm, out_hbm, idx_v, rows_a, rows_b,
           sem_a, sem_b, osem_a, osem_b):
        wid = lax.axis_index("s") * nc + lax.axis_index("c")
        base = wid * b_per_w
        pltpu.sync_copy(idx_hbm.at[pl.ds(base, b_per_w)], idx_v)

        def body(i, carry):
            c0 = i * (2 * chunk)
            ga = pltpu.async_copy(
                table_hbm.at[idx_v.at[pl.ds(c0, chunk)]], rows_a, sem_a)
            gb = pltpu.async_copy(
                table_hbm.at[idx_v.at[pl.ds(c0 + chunk, chunk)]], rows_b, sem_b)
            ga.wait()
            wa = pltpu.async_copy(rows_a, out_hbm.at[pl.ds(base + c0, chunk)],
                                  osem_a)
            gb.wait()
            wb = pltpu.async_copy(rows_b,
                                  out_hbm.at[pl.ds(base + c0 + chunk, chunk)],
                                  osem_b)
            wa.wait()
            wb.wait()
            return carry

        lax.fori_loop(0, b_per_w // (2 * chunk), body, 0)

    return gk(table, idx)


def _leaky(x):
    return jnp.where(x > 0, x, 0.1 * x)


def _mlp_body(*refs, d_in, nlayers, nsample):
    p_ref, x_ref, g_ref = refs[0], refs[1], refs[2]
    w0_ref, b0_ref = refs[3], refs[4]
    lrefs = refs[5:5 + 2 * nlayers]
    o_ref = refs[5 + 2 * nlayers]
    w0 = w0_ref[...]
    wa = w0[:, :d_in]
    wc = w0[:, 2 * d_in:]
    q = (lax.dot_general(p_ref[...], wa, (((1,), (1,)), ((), ())),
                         preferred_element_type=jnp.float32)
         - lax.dot_general(x_ref[...], wc, (((1,), (1,)), ((), ())),
                           preferred_element_type=jnp.float32)
         + b0_ref[...])
    t, cout = q.shape
    g = g_ref[...].astype(jnp.float32)
    h3 = g.reshape(t, nsample, cout) + q[:, None, :]
    h = _leaky(h3).reshape(t * nsample, cout)
    for li in range(nlayers):
        w = lrefs[2 * li][...].astype(jnp.bfloat16)
        b = lrefs[2 * li + 1][...]
        h = _leaky(lax.dot_general(h.astype(jnp.bfloat16), w,
                                   (((1,), (1,)), ((), ())),
                                   preferred_element_type=jnp.float32) + b)
    o_ref[...] = jnp.max(h.reshape(t, nsample, cout), axis=1)


def _mlp(pts_q, xyz_q, g, w0, b0, layers, nsample):
    m, d_in = pts_q.shape
    cout = w0.shape[0]
    t = _MLP_T
    nlayers = len(layers)
    operands = [pts_q, xyz_q, g, w0, b0.reshape(1, cout)]
    in_specs = [
        pl.BlockSpec((t, d_in), lambda i: (i, 0)),
        pl.BlockSpec((t, 3), lambda i: (i, 0)),
        pl.BlockSpec((t * nsample, cout), lambda i: (i, 0)),
        pl.BlockSpec(w0.shape, lambda i: (0, 0)),
        pl.BlockSpec((1, cout), lambda i: (0, 0)),
    ]
    for w, b in layers:
        operands += [w, b.reshape(1, cout)]
        in_specs += [pl.BlockSpec(w.shape, lambda i: (0, 0)),
                     pl.BlockSpec((1, cout), lambda i: (0, 0))]
    return pl.pallas_call(
        functools.partial(_mlp_body, d_in=d_in, nlayers=nlayers,
                          nsample=nsample),
        grid=(m // t,),
        in_specs=in_specs,
        out_specs=pl.BlockSpec((t, cout), lambda i: (i, 0)),
        out_shape=jax.ShapeDtypeStruct((m, cout), jnp.float32),
    )(*operands)


def kernel(pc1, pc2, feat1, feat2, W1_0, b1_0, W1_1, b1_1, W1_2, b1_2,
           W2_0, b2_0, W2_1, b2_1):
    b, _, n = pc1.shape
    c = feat1.shape[1]
    k = NSAMPLE
    p1 = jnp.transpose(pc1, (0, 2, 1))
    p2 = jnp.transpose(pc2, (0, 2, 1))
    f1 = jnp.transpose(feat1, (0, 2, 1))
    f2 = jnp.transpose(feat2, (0, 2, 1))

    spts = jnp.concatenate([f2, f1], 0).reshape(2 * b * n, c)
    sxyz = jnp.concatenate([p2, p1], 0).reshape(2 * b * n, 3)
    s12 = _proj(spts, sxyz, W1_0)

    offs = (jnp.arange(b, dtype=jnp.int32) * n)[:, None, None]
    idx1 = _knn(p2, pc1, k)
    g1 = _gather_rows(s12, (idx1 + offs + b * n).reshape(-1))
    idx0 = _knn(p1, pc2, k)
    g0 = _gather_rows(s12, (idx0 + offs).reshape(-1))
    out1 = _mlp(f2.reshape(b * n, c), p2.reshape(b * n, 3), g1,
                W1_0, b1_0, [(W1_1, b1_1), (W1_2, b1_2)], k)
    feat2_new = out1.reshape(b, n, -1)

    cn = feat2_new.shape[-1]
    s3 = _proj(feat2_new.reshape(b * n, cn), p2.reshape(b * n, 3), W2_0)
    g3 = _gather_rows(s3, (idx0 + offs).reshape(-1))
    out0 = _mlp(f1.reshape(b * n, c), p1.reshape(b * n, 3), g0,
                W1_0, b1_0, [(W1_1, b1_1), (W1_2, b1_2)], k)
    feat1_new = out0.reshape(b, n, -1)
    out3 = _mlp(feat1_new.reshape(b * n, cn), p1.reshape(b * n, 3), g3,
                W2_0, b2_0, [(W2_1, b2_1)], k)

    to_cn = lambda x: jnp.transpose(x, (0, 2, 1))
    return (to_cn(feat1_new), to_cn(feat2_new),
            to_cn(out3.reshape(b, n, -1)))
